# initial kernel scaffold (unmeasured)
import jax
import jax.numpy as jnp
from jax import lax
from jax.experimental import pallas as pl
from jax.experimental.pallas import tpu as pltpu

ZDIM = 4
B, H, D, BS = 16, 16, 64, 16
NBT = 128
LANES = 128
NEG = -1e30


def kernel(Q, K, V, bt, lens):
    nloc = K.shape[0]
    nk = nloc * BS

    def body(q_ref, k_ref, v_ref, bt_ref, lens_ref, out_ref,
             comm_ref, send_sems, recv_sems):
        my_x = lax.axis_index("x")
        my_y = lax.axis_index("y")
        my_z = lax.axis_index("z")
        base = my_z * nloc

        q = q_ref[...].reshape(B, H, D).astype(jnp.bfloat16)
        k = k_ref[...].reshape(nk, H, D).astype(jnp.bfloat16)
        v = v_ref[...].reshape(nk, H, D).astype(jnp.bfloat16)

        s = lax.dot_general(
            q, k, (((2,), (2,)), ((1,), (1,))),
            preferred_element_type=jnp.float32,
        ) * (D ** -0.5)

        btv = bt_ref[...]
        lensv = lens_ref[...]
        jidx = lax.broadcasted_iota(jnp.int32, (B, NBT, nloc), 1)
        pidx = lax.broadcasted_iota(jnp.int32, (B, NBT, nloc), 2)
        hits = (btv[:, :, None] == base + pidx) & (jidx < lensv[:, :, None])
        cnt = jnp.sum(hits.astype(jnp.float32), axis=1)
        cntk = jnp.broadcast_to(
            cnt[:, :, None], (B, nloc, BS)).reshape(B, nk)

        s = jnp.where(cntk[None] > 0, s, NEG)
        m = jnp.max(s, axis=-1)
        e = jnp.exp(s - m[:, :, None]) * cntk[None]
        l = jnp.sum(e, axis=-1)
        o = lax.dot_general(
            e.astype(jnp.bfloat16), v, (((2,), (0,)), ((0,), (1,))),
            preferred_element_type=jnp.float32,
        )

        comm_ref[ZDIM - 1, :, :, 0:D] = o
        comm_ref[ZDIM - 1, :, :, D:D + 1] = m[:, :, None]
        comm_ref[ZDIM - 1, :, :, D + 1:D + 2] = l[:, :, None]

        barrier = pltpu.get_barrier_semaphore()
        for dz in (1, ZDIM - 1):
            pl.semaphore_signal(
                barrier, inc=1,
                device_id=(my_x, my_y, (my_z + dz) % ZDIM),
                device_id_type=pl.DeviceIdType.MESH,
            )
        pl.semaphore_wait(barrier, 2)

        right = (my_x, my_y, (my_z + 1) % ZDIM)
        for h in range(ZDIM - 1):
            src_slot = (ZDIM - 1) if h == 0 else h - 1
            rdma = pltpu.make_async_remote_copy(
                src_ref=comm_ref.at[src_slot],
                dst_ref=comm_ref.at[h],
                send_sem=send_sems.at[h],
                recv_sem=recv_sems.at[h],
                device_id=right,
                device_id_type=pl.DeviceIdType.MESH,
            )
            rdma.start()
            rdma.wait()

        ms = [comm_ref[i, :, :, D:D + 1] for i in range(ZDIM)]
        mx = ms[0]
        for mi in ms[1:]:
            mx = jnp.maximum(mx, mi)
        acc_o = jnp.zeros((H, B, D), jnp.float32)
        acc_l = jnp.zeros((H, B, 1), jnp.float32)
        for i in range(ZDIM):
            alpha = jnp.exp(ms[i] - mx)
            acc_o = acc_o + comm_ref[i, :, :, 0:D] * alpha
            acc_l = acc_l + comm_ref[i, :, :, D + 1:D + 2] * alpha
        res = acc_o / acc_l
        out_ref[...] = res.transpose(1, 0, 2).reshape(B, 1, H, D)

    return pl.pallas_call(
        body,
        out_shape=jax.ShapeDtypeStruct((B, 1, H, D), jnp.float32),
        in_specs=[pl.BlockSpec(memory_space=pltpu.VMEM)] * 5,
        out_specs=pl.BlockSpec(memory_space=pltpu.VMEM),
        scratch_shapes=[
            pltpu.VMEM((ZDIM, H, B, LANES), jnp.float32),
            pltpu.SemaphoreType.DMA((ZDIM - 1,)),
            pltpu.SemaphoreType.DMA((ZDIM - 1,)),
        ],
        compiler_params=pltpu.CompilerParams(collective_id=0),
    )(Q, K, V, bt, lens.reshape(B, 1))


# baseline (device time: 97294 ns/iter reference)
import jax
import jax.numpy as jnp
from jax import lax
from jax.experimental import pallas as pl
from jax.experimental.pallas import tpu as pltpu

ZDIM = 4
B, H, D, BS = 16, 16, 64, 16
NBT = 128
LANES = 128
NEG = -1e30


def kernel(Q, K, V, bt, lens):
    nloc = K.shape[0]
    nk = nloc * BS

    def body(q_ref, k_ref, v_ref, bt_ref, lens_ref, out_ref,
             comm_ref, send_sems, recv_sems):
        my_x = lax.axis_index("x")
        my_y = lax.axis_index("y")
        my_z = lax.axis_index("z")
        base = my_z * nloc

        btv = bt_ref[...]
        lensv = lens_ref[...]
        jidx = lax.broadcasted_iota(jnp.int32, (B, NBT, nloc), 1)
        pidx = lax.broadcasted_iota(jnp.int32, (B, NBT, nloc), 2)
        hits = (btv[:, :, None] == base + pidx) & (jidx < lensv[:, :, None])
        cnt = jnp.sum(hits.astype(jnp.float32), axis=1)
        cntk = jnp.broadcast_to(
            cnt[:, :, None], (B, nloc, BS)).reshape(B, nk)

        for h in range(H):
            qh = q_ref[:, 0, h, :].astype(jnp.bfloat16)
            kh = k_ref[:, :, h, :].reshape(nk, D).astype(jnp.bfloat16)
            vh = v_ref[:, :, h, :].reshape(nk, D).astype(jnp.bfloat16)
            s = lax.dot_general(
                qh, kh, (((1,), (1,)), ((), ())),
                preferred_element_type=jnp.float32,
            ) * (D ** -0.5)
            s = jnp.where(cntk > 0, s, NEG)
            m = jnp.max(s, axis=-1, keepdims=True)
            e = jnp.exp(s - m) * cntk
            l = jnp.sum(e, axis=-1, keepdims=True)
            o = lax.dot_general(
                e.astype(jnp.bfloat16), vh, (((1,), (0,)), ((), ())),
                preferred_element_type=jnp.float32,
            )
            comm_ref[ZDIM - 1, h, :, 0:D] = o
            comm_ref[ZDIM - 1, h, :, D:D + 1] = m
            comm_ref[ZDIM - 1, h, :, D + 1:D + 2] = l

        barrier = pltpu.get_barrier_semaphore()
        for dz in (1, ZDIM - 1):
            pl.semaphore_signal(
                barrier, inc=1,
                device_id=(my_x, my_y, (my_z + dz) % ZDIM),
                device_id_type=pl.DeviceIdType.MESH,
            )
        pl.semaphore_wait(barrier, 2)

        right = (my_x, my_y, (my_z + 1) % ZDIM)
        for h in range(ZDIM - 1):
            src_slot = (ZDIM - 1) if h == 0 else h - 1
            rdma = pltpu.make_async_remote_copy(
                src_ref=comm_ref.at[src_slot],
                dst_ref=comm_ref.at[h],
                send_sem=send_sems.at[h],
                recv_sem=recv_sems.at[h],
                device_id=right,
                device_id_type=pl.DeviceIdType.MESH,
            )
            rdma.start()
            rdma.wait()

        ms = [comm_ref[i, :, :, D:D + 1] for i in range(ZDIM)]
        mx = ms[0]
        for mi in ms[1:]:
            mx = jnp.maximum(mx, mi)
        acc_o = jnp.zeros((H, B, D), jnp.float32)
        acc_l = jnp.zeros((H, B, 1), jnp.float32)
        for i in range(ZDIM):
            alpha = jnp.exp(ms[i] - mx)
            acc_o = acc_o + comm_ref[i, :, :, 0:D] * alpha
            acc_l = acc_l + comm_ref[i, :, :, D + 1:D + 2] * alpha
        res = acc_o / acc_l
        out_ref[...] = res.transpose(1, 0, 2).reshape(B, 1, H, D)

    return pl.pallas_call(
        body,
        out_shape=jax.ShapeDtypeStruct((B, 1, H, D), jnp.float32),
        in_specs=[pl.BlockSpec(memory_space=pltpu.VMEM)] * 5,
        out_specs=pl.BlockSpec(memory_space=pltpu.VMEM),
        scratch_shapes=[
            pltpu.VMEM((ZDIM, H, B, LANES), jnp.float32),
            pltpu.SemaphoreType.DMA((ZDIM - 1,)),
            pltpu.SemaphoreType.DMA((ZDIM - 1,)),
        ],
        compiler_params=pltpu.CompilerParams(
            collective_id=0, vmem_limit_bytes=100 * 1024 * 1024,
        ),
    )(Q, K, V, bt, lens.reshape(B, 1))


# device time: 70900 ns/iter; 1.3723x vs baseline; 1.3723x over previous
import jax
import jax.numpy as jnp
from jax import lax
from jax.experimental import pallas as pl
from jax.experimental.pallas import tpu as pltpu

ZDIM = 4
B, H, D, BS = 16, 16, 64, 16
NBT = 128
LANES = 128
NEG = -1e30


def kernel(Q, K, V, bt, lens):
    nloc = K.shape[0]
    nk = nloc * BS

    def body(q_ref, k_ref, v_ref, bt_ref, lens_ref, out_ref,
             comm_ref, kt_ref, vt_ref, send_sems, recv_sems):
        my_x = lax.axis_index("x")
        my_y = lax.axis_index("y")
        my_z = lax.axis_index("z")
        base = my_z * nloc

        btv = bt_ref[...]
        lensv = lens_ref[...]
        jidx = lax.broadcasted_iota(jnp.int32, (B, NBT, nloc), 1)
        pidx = lax.broadcasted_iota(jnp.int32, (B, NBT, nloc), 2)
        hits = (btv[:, :, None] == base + pidx) & (jidx < lensv[:, :, None])
        cnt = jnp.sum(hits.astype(jnp.float32), axis=1)
        cntk = jnp.broadcast_to(
            cnt[:, :, None], (B, nloc, BS)).reshape(B, nk)

        kt_ref[...] = k_ref[...].reshape(nk, H, D).astype(
            jnp.bfloat16).transpose(1, 0, 2)
        vt_ref[...] = v_ref[...].reshape(nk, H, D).astype(
            jnp.bfloat16).transpose(1, 0, 2)

        for h in range(H):
            qh = q_ref[:, 0, h, :].astype(jnp.bfloat16)
            kh = kt_ref[h]
            vh = vt_ref[h]
            s = lax.dot_general(
                qh, kh, (((1,), (1,)), ((), ())),
                preferred_element_type=jnp.float32,
            ) * (D ** -0.5)
            s = jnp.where(cntk > 0, s, NEG)
            m = jnp.max(s, axis=-1, keepdims=True)
            e = jnp.exp(s - m) * cntk
            l = jnp.sum(e, axis=-1, keepdims=True)
            o = lax.dot_general(
                e.astype(jnp.bfloat16), vh, (((1,), (0,)), ((), ())),
                preferred_element_type=jnp.float32,
            )
            comm_ref[ZDIM - 1, h, :, 0:D] = o
            comm_ref[ZDIM - 1, h, :, D:D + 1] = m
            comm_ref[ZDIM - 1, h, :, D + 1:D + 2] = l

        barrier = pltpu.get_barrier_semaphore()
        for dz in (1, ZDIM - 1):
            pl.semaphore_signal(
                barrier, inc=1,
                device_id=(my_x, my_y, (my_z + dz) % ZDIM),
                device_id_type=pl.DeviceIdType.MESH,
            )
        pl.semaphore_wait(barrier, 2)

        right = (my_x, my_y, (my_z + 1) % ZDIM)
        for h in range(ZDIM - 1):
            src_slot = (ZDIM - 1) if h == 0 else h - 1
            rdma = pltpu.make_async_remote_copy(
                src_ref=comm_ref.at[src_slot],
                dst_ref=comm_ref.at[h],
                send_sem=send_sems.at[h],
                recv_sem=recv_sems.at[h],
                device_id=right,
                device_id_type=pl.DeviceIdType.MESH,
            )
            rdma.start()
            rdma.wait()

        ms = [comm_ref[i, :, :, D:D + 1] for i in range(ZDIM)]
        mx = ms[0]
        for mi in ms[1:]:
            mx = jnp.maximum(mx, mi)
        acc_o = jnp.zeros((H, B, D), jnp.float32)
        acc_l = jnp.zeros((H, B, 1), jnp.float32)
        for i in range(ZDIM):
            alpha = jnp.exp(ms[i] - mx)
            acc_o = acc_o + comm_ref[i, :, :, 0:D] * alpha
            acc_l = acc_l + comm_ref[i, :, :, D + 1:D + 2] * alpha
        res = acc_o / acc_l
        out_ref[...] = res.transpose(1, 0, 2).reshape(B, 1, H, D)

    return pl.pallas_call(
        body,
        out_shape=jax.ShapeDtypeStruct((B, 1, H, D), jnp.float32),
        in_specs=[pl.BlockSpec(memory_space=pltpu.VMEM)] * 5,
        out_specs=pl.BlockSpec(memory_space=pltpu.VMEM),
        scratch_shapes=[
            pltpu.VMEM((ZDIM, H, B, LANES), jnp.float32),
            pltpu.VMEM((H, nk, D), jnp.bfloat16),
            pltpu.VMEM((H, nk, D), jnp.bfloat16),
            pltpu.SemaphoreType.DMA((ZDIM - 1,)),
            pltpu.SemaphoreType.DMA((ZDIM - 1,)),
        ],
        compiler_params=pltpu.CompilerParams(
            collective_id=0, vmem_limit_bytes=100 * 1024 * 1024,
        ),
    )(Q, K, V, bt, lens.reshape(B, 1))


# device time: 69258 ns/iter; 1.4048x vs baseline; 1.0237x over previous
import jax
import jax.numpy as jnp
from jax import lax
from jax.experimental import pallas as pl
from jax.experimental.pallas import tpu as pltpu

ZDIM = 4
B, H, D, BS = 16, 16, 64, 16
NBT = 128
LANES = 128
NEG = -1e30


def kernel(Q, K, V, bt, lens):
    nloc = K.shape[0]
    nk = nloc * BS

    def body(q_ref, k_ref, v_ref, bt_ref, lens_ref, out_ref,
             comm_ref, kt_ref, vt_ref, send_sems, recv_sems):
        my_x = lax.axis_index("x")
        my_y = lax.axis_index("y")
        my_z = lax.axis_index("z")
        base = my_z * nloc

        btv = bt_ref[...]
        lensv = lens_ref[...]
        jidx = lax.broadcasted_iota(jnp.int32, (B, NBT, nloc), 1)
        pidx = lax.broadcasted_iota(jnp.int32, (B, NBT, nloc), 2)
        hits = (btv[:, :, None] == base + pidx) & (jidx < lensv[:, :, None])
        cnt = jnp.sum(hits.astype(jnp.float32), axis=1)
        cntk = jnp.broadcast_to(
            cnt[:, :, None], (B, nloc, BS)).reshape(B, nk)

        kt_ref[...] = k_ref[...].reshape(nk, H, D).astype(
            jnp.bfloat16).transpose(1, 0, 2)
        vt_ref[...] = v_ref[...].reshape(nk, H, D).astype(
            jnp.bfloat16).transpose(1, 0, 2)

        q_all = q_ref[...].reshape(B, H, D).astype(jnp.bfloat16)
        s = lax.dot_general(
            q_all, kt_ref[...], (((2,), (2,)), ((1,), (0,))),
            preferred_element_type=jnp.float32,
        ) * (D ** -0.5)
        s = jnp.where(cntk[None] > 0, s, NEG)
        m = jnp.max(s, axis=-1, keepdims=True)
        e = jnp.exp(s - m) * cntk[None]
        l = jnp.sum(e, axis=-1, keepdims=True)
        o = lax.dot_general(
            e.astype(jnp.bfloat16), vt_ref[...], (((2,), (1,)), ((0,), (0,))),
            preferred_element_type=jnp.float32,
        )
        comm_ref[ZDIM - 1, :, :, 0:D] = o
        comm_ref[ZDIM - 1, :, :, D:D + 1] = m
        comm_ref[ZDIM - 1, :, :, D + 1:D + 2] = l

        barrier = pltpu.get_barrier_semaphore()
        for dz in (1, ZDIM - 1):
            pl.semaphore_signal(
                barrier, inc=1,
                device_id=(my_x, my_y, (my_z + dz) % ZDIM),
                device_id_type=pl.DeviceIdType.MESH,
            )
        pl.semaphore_wait(barrier, 2)

        right = (my_x, my_y, (my_z + 1) % ZDIM)
        for h in range(ZDIM - 1):
            src_slot = (ZDIM - 1) if h == 0 else h - 1
            rdma = pltpu.make_async_remote_copy(
                src_ref=comm_ref.at[src_slot],
                dst_ref=comm_ref.at[h],
                send_sem=send_sems.at[h],
                recv_sem=recv_sems.at[h],
                device_id=right,
                device_id_type=pl.DeviceIdType.MESH,
            )
            rdma.start()
            rdma.wait()

        ms = [comm_ref[i, :, :, D:D + 1] for i in range(ZDIM)]
        mx = ms[0]
        for mi in ms[1:]:
            mx = jnp.maximum(mx, mi)
        acc_o = jnp.zeros((H, B, D), jnp.float32)
        acc_l = jnp.zeros((H, B, 1), jnp.float32)
        for i in range(ZDIM):
            alpha = jnp.exp(ms[i] - mx)
            acc_o = acc_o + comm_ref[i, :, :, 0:D] * alpha
            acc_l = acc_l + comm_ref[i, :, :, D + 1:D + 2] * alpha
        res = acc_o / acc_l
        out_ref[...] = res.transpose(1, 0, 2).reshape(B, 1, H, D)

    return pl.pallas_call(
        body,
        out_shape=jax.ShapeDtypeStruct((B, 1, H, D), jnp.float32),
        in_specs=[pl.BlockSpec(memory_space=pltpu.VMEM)] * 5,
        out_specs=pl.BlockSpec(memory_space=pltpu.VMEM),
        scratch_shapes=[
            pltpu.VMEM((ZDIM, H, B, LANES), jnp.float32),
            pltpu.VMEM((H, nk, D), jnp.bfloat16),
            pltpu.VMEM((H, nk, D), jnp.bfloat16),
            pltpu.SemaphoreType.DMA((ZDIM - 1,)),
            pltpu.SemaphoreType.DMA((ZDIM - 1,)),
        ],
        compiler_params=pltpu.CompilerParams(
            collective_id=0, vmem_limit_bytes=100 * 1024 * 1024,
        ),
    )(Q, K, V, bt, lens.reshape(B, 1))


# device time: 60475 ns/iter; 1.6088x vs baseline; 1.1452x over previous
import jax
import jax.numpy as jnp
from jax import lax
from jax.experimental import pallas as pl
from jax.experimental.pallas import tpu as pltpu

ZDIM = 4
B, H, D, BS = 16, 16, 64, 16
NBT = 128
LANES = 128
NEG = -1e30


def kernel(Q, K, V, bt, lens):
    nloc = K.shape[0]
    nk = nloc * BS

    def body(q_ref, k_ref, v_ref, bt_ref, lens_ref, out_ref,
             comm_ref, kt_ref, vt_ref, send_sems, recv_sems):
        my_x = lax.axis_index("x")
        my_y = lax.axis_index("y")
        my_z = lax.axis_index("z")
        base = my_z * nloc

        with jax.named_scope("count"):
            btv = bt_ref[...]
            lensv = lens_ref[...]
            jidx = lax.broadcasted_iota(jnp.int32, (B, NBT, nloc), 1)
            pidx = lax.broadcasted_iota(jnp.int32, (B, NBT, nloc), 2)
            hits = (btv[:, :, None] == base + pidx) & (jidx < lensv[:, :, None])
            cnt = jnp.sum(hits.astype(jnp.float32), axis=1)
            cntk = jnp.broadcast_to(
                cnt[:, :, None], (B, nloc, BS)).reshape(B, nk)

        with jax.named_scope("relayout"):
            kt_ref[...] = k_ref[...].reshape(nk, H, D).transpose(1, 0, 2)
            vt_ref[...] = v_ref[...].reshape(nk, H, D).transpose(1, 0, 2)

        with jax.named_scope("attn"):
            q_all = q_ref[...].reshape(B, H, D)
            s = lax.dot_general(
                q_all, kt_ref[...], (((2,), (2,)), ((1,), (0,))),
                preferred_element_type=jnp.float32,
            ) * (D ** -0.5)
            s = jnp.where(cntk[None] > 0, s, NEG)
            m = jnp.max(s, axis=-1, keepdims=True)
            e = jnp.exp(s - m) * cntk[None]
            l = jnp.sum(e, axis=-1, keepdims=True)
            o = lax.dot_general(
                e.astype(jnp.bfloat16), vt_ref[...],
                (((2,), (1,)), ((0,), (0,))),
                preferred_element_type=jnp.float32,
            )

        with jax.named_scope("pack"):
            comm_ref[0, :, :, 0:D] = o
            comm_ref[0, :, :, D:D + 1] = m
            comm_ref[0, :, :, D + 1:D + 2] = l

        with jax.named_scope("barrier"):
            barrier = pltpu.get_barrier_semaphore()
            for dz in range(1, ZDIM):
                pl.semaphore_signal(
                    barrier, inc=1,
                    device_id=(my_x, my_y, (my_z + dz) % ZDIM),
                    device_id_type=pl.DeviceIdType.MESH,
                )
            pl.semaphore_wait(barrier, ZDIM - 1)

        with jax.named_scope("a2a_start"):
            rdmas = []
            for dz in range(1, ZDIM):
                rdma = pltpu.make_async_remote_copy(
                    src_ref=comm_ref.at[0],
                    dst_ref=comm_ref.at[dz],
                    send_sem=send_sems.at[dz - 1],
                    recv_sem=recv_sems.at[dz - 1],
                    device_id=(my_x, my_y, (my_z + dz) % ZDIM),
                    device_id_type=pl.DeviceIdType.MESH,
                )
                rdma.start()
                rdmas.append(rdma)
        with jax.named_scope("a2a_wait"):
            for rdma in rdmas:
                rdma.wait()

        with jax.named_scope("merge"):
            ms = [comm_ref[i, :, :, D:D + 1] for i in range(ZDIM)]
            mx = ms[0]
            for mi in ms[1:]:
                mx = jnp.maximum(mx, mi)
            acc_o = jnp.zeros((H, B, D), jnp.float32)
            acc_l = jnp.zeros((H, B, 1), jnp.float32)
            for i in range(ZDIM):
                alpha = jnp.exp(ms[i] - mx)
                acc_o = acc_o + comm_ref[i, :, :, 0:D] * alpha
                acc_l = acc_l + comm_ref[i, :, :, D + 1:D + 2] * alpha
            res = acc_o / acc_l
            out_ref[...] = res.transpose(1, 0, 2).reshape(B, 1, H, D)

    return pl.pallas_call(
        body,
        out_shape=jax.ShapeDtypeStruct((B, 1, H, D), jnp.float32),
        in_specs=[pl.BlockSpec(memory_space=pltpu.VMEM)] * 5,
        out_specs=pl.BlockSpec(memory_space=pltpu.VMEM),
        scratch_shapes=[
            pltpu.VMEM((ZDIM, H, B, LANES), jnp.float32),
            pltpu.VMEM((H, nk, D), jnp.bfloat16),
            pltpu.VMEM((H, nk, D), jnp.bfloat16),
            pltpu.SemaphoreType.DMA((ZDIM - 1,)),
            pltpu.SemaphoreType.DMA((ZDIM - 1,)),
        ],
        compiler_params=pltpu.CompilerParams(
            collective_id=0, vmem_limit_bytes=100 * 1024 * 1024,
        ),
    )(Q.astype(jnp.bfloat16), K.astype(jnp.bfloat16),
      V.astype(jnp.bfloat16), bt, lens.reshape(B, 1))
